# Initial kernel scaffold; baseline (speedup 1.0000x reference)
import jax
import jax.numpy as jnp
from jax.experimental import pallas as pl


def kernel(residual, emb):
    raise NotImplementedError("write your pallas kernel here")


# speedup vs baseline: 2.3637x; 2.3637x over previous
import jax
import jax.numpy as jnp
from jax.experimental import pallas as pl

N_TOK = 2048
K = 1024
D = 64
TOK_BLK = 256


def _group_sum(x):
    return ((x[0] + x[4]) + (x[2] + x[6])) + ((x[1] + x[5]) + (x[3] + x[7]))


def _vq_block(r_ref, embt_ref, emb_ref, q_ref, codes_ref):
    r = r_ref[...]
    embt = embt_ref[...]
    acc = jnp.zeros((TOK_BLK, K), jnp.float32)
    for v in range(D // 8):
        sq = []
        for s in range(8):
            d = 8 * v + s
            diff = r[:, d:d + 1] - embt[d:d + 1, :]
            sq.append(diff * diff)
        acc = acc + _group_sum(sq)
    dist = acc
    minval = jnp.min(dist, axis=1, keepdims=True)
    iota = jax.lax.broadcasted_iota(jnp.int32, dist.shape, 1)
    codes = jnp.min(jnp.where(dist == minval, iota, K), axis=1, keepdims=True)
    codes_ref[...] = codes
    onehot = (codes == iota).astype(jnp.float32)
    q_ref[...] = jnp.dot(onehot, emb_ref[...],
                         preferred_element_type=jnp.float32,
                         precision=jax.lax.Precision.HIGHEST)


def kernel(residual, emb):
    embt = emb.T
    grid = (N_TOK // TOK_BLK,)
    q, codes = pl.pallas_call(
        _vq_block,
        grid=grid,
        in_specs=[
            pl.BlockSpec((TOK_BLK, D), lambda i: (i, 0)),
            pl.BlockSpec((D, K), lambda i: (0, 0)),
            pl.BlockSpec((K, D), lambda i: (0, 0)),
        ],
        out_specs=[
            pl.BlockSpec((TOK_BLK, D), lambda i: (i, 0)),
            pl.BlockSpec((TOK_BLK, 1), lambda i: (i, 0)),
        ],
        out_shape=[
            jax.ShapeDtypeStruct((N_TOK, D), jnp.float32),
            jax.ShapeDtypeStruct((N_TOK, 1), jnp.int32),
        ],
    )(residual, embt, emb)
    return (q, codes.reshape(N_TOK))


# speedup vs baseline: 3.0331x; 1.2832x over previous
import functools

import jax
import jax.numpy as jnp
from jax import lax
from jax.experimental import pallas as pl
from jax.experimental.pallas import tpu as pltpu
from jax.experimental.pallas import tpu_sc as plsc

N_TOK = 2048
K = 1024
D = 64
J = 8
F = N_TOK * J
NW = 32
B_PER_W = F // NW
DPAD = 128
SCALE = float(1 << 20)
IMAX = 2147483647


def _topj_kernel(r_ref, embt_ref, cand_ref):
    r = r_ref[...]
    embt = embt_ref[...]
    dots = jnp.dot(r, embt, preferred_element_type=jnp.float32)
    e2 = jnp.sum(embt * embt, axis=0, keepdims=True)
    dist = e2 - 2.0 * dots
    iota = jax.lax.broadcasted_iota(jnp.int32, dist.shape, 1)
    packed = (dist * SCALE).astype(jnp.int32) * K + iota
    for j in range(J):
        m = jnp.min(packed, axis=1, keepdims=True)
        cand_ref[:, j:j + 1] = m & (K - 1)
        packed = jnp.where(packed == m, IMAX, packed)


def _topj(residual, embt):
    return pl.pallas_call(
        _topj_kernel,
        out_shape=jax.ShapeDtypeStruct((N_TOK, J), jnp.int32),
    )(residual, embt)


def _sc_gather_kernel(emb_hbm, idx_hbm, out_hbm, idx_v, rows_v, sem):
    wid = lax.axis_index("s") * 2 + lax.axis_index("c")
    base = wid * B_PER_W
    pltpu.sync_copy(idx_hbm.at[pl.ds(base, B_PER_W)], idx_v)
    pltpu.async_copy(emb_hbm.at[idx_v], rows_v, sem).wait()
    pltpu.sync_copy(rows_v, out_hbm.at[pl.ds(base, B_PER_W)])


def _sc_gather(emb_padded, cand_flat):
    mesh = plsc.VectorSubcoreMesh(core_axis_name="c", subcore_axis_name="s")
    fn = functools.partial(
        pl.kernel,
        mesh=mesh,
        out_type=jax.ShapeDtypeStruct((F, DPAD), jnp.float32),
        scratch_types=[
            pltpu.VMEM((B_PER_W,), jnp.int32),
            pltpu.VMEM((B_PER_W, DPAD), jnp.float32),
            pltpu.SemaphoreType.DMA,
        ],
    )(_sc_gather_kernel)
    return fn(emb_padded, cand_flat)


def _rescore_kernel(r_ref, rows_ref, idx_ref, q_ref, codes_ref):
    r = r_ref[...]
    best_d = None
    best_k = None
    best_j = None
    for j in range(J):
        slab = rows_ref[j * N_TOK:(j + 1) * N_TOK, 0:D]
        diff = r - slab
        sq = (diff * diff).T
        dist_j = None
        for v in range(D // 8):
            g = sq[8 * v:8 * v + 8, :]
            a = g[0:4, :] + g[4:8, :]
            b = a[0:2, :] + a[2:4, :]
            gv = b[0:1, :] + b[1:2, :]
            dist_j = gv if dist_j is None else dist_j + gv
        k_j = idx_ref[j:j + 1, :]
        if j == 0:
            best_d, best_k = dist_j, k_j
            best_j = jnp.zeros_like(k_j)
        else:
            take = (dist_j < best_d) | ((dist_j == best_d) & (k_j < best_k))
            best_d = jnp.where(take, dist_j, best_d)
            best_k = jnp.where(take, k_j, best_k)
            best_j = jnp.where(take, jnp.int32(j), best_j)
    codes_ref[...] = best_k
    best_j_col = best_j.T
    q = rows_ref[0:N_TOK, 0:D]
    for j in range(1, J):
        slab = rows_ref[j * N_TOK:(j + 1) * N_TOK, 0:D]
        q = jnp.where(best_j_col == j, slab, q)
    q_ref[...] = q


def _rescore(residual, rows, cand_idx):
    return pl.pallas_call(
        _rescore_kernel,
        out_shape=[
            jax.ShapeDtypeStruct((N_TOK, D), jnp.float32),
            jax.ShapeDtypeStruct((1, N_TOK), jnp.int32),
        ],
    )(residual, rows, cand_idx)


def kernel(residual, emb):
    embt = emb.T
    cand = _topj(residual, embt)
    cand_jmajor = cand.T
    cand_flat = cand_jmajor.reshape(F)
    emb_padded = jnp.pad(emb, ((0, 0), (0, DPAD - D)))
    rows = _sc_gather(emb_padded, cand_flat)
    q, codes = _rescore(residual, rows, cand_jmajor)
    return (q, codes.reshape(N_TOK))
